# baseline (speedup 1.0000x reference)
import jax
import jax.numpy as jnp
from jax import lax
from jax.experimental import pallas as pl
from jax.experimental.pallas import tpu as pltpu
from jax.experimental.pallas import tpu_sc as plsc

_B = 16
_K = 8192
_KNN = 16
_L = 16
_NROWS = _KNN // 2


def _table_body(advt_ref, orit_ref, out_ref):
    d = advt_ref[...] - orit_ref[...]
    out_ref[...] = jnp.sum(d * d, axis=0)


def _worker_body(tab_h, idx_h, w_h, out0_h, out1_h,
                 idx0, idx1, table, wv, outv, semt, sem0, sem1):
    c = lax.axis_index("c")
    s = lax.axis_index("s")
    b = s
    nbase = c * _NROWS

    ht = pltpu.async_copy(tab_h.at[b], table, semt)

    bufs = (idx0, idx1)
    sems = (sem0, sem1)

    def start_chunk(ci, slot):
        return pltpu.async_copy(idx_h.at[b, nbase + ci], bufs[slot], sems[slot])

    h = [start_chunk(0, 0), start_chunk(1, 1)]
    pltpu.sync_copy(w_h.at[b], wv)
    ht.wait()

    zero = jnp.zeros((_L,), jnp.float32)
    accs = (zero, zero, zero, zero)

    def make_gbody(buf):
        def gbody(i, a):
            base = pl.multiple_of(i * 4 * _L, 4 * _L)
            r0 = a[0] + plsc.load_gather(table, [buf[pl.ds(base, _L)]])
            r1 = a[1] + plsc.load_gather(table, [buf[pl.ds(base + _L, _L)]])
            r2 = a[2] + plsc.load_gather(table, [buf[pl.ds(base + 2 * _L, _L)]])
            r3 = a[3] + plsc.load_gather(table, [buf[pl.ds(base + 3 * _L, _L)]])
            return (r0, r1, r2, r3)
        return gbody

    for ci in range(_NROWS):
        slot = ci % 2
        h[slot].wait()
        accs = lax.fori_loop(0, _K // (4 * _L), make_gbody(bufs[slot]),
                             accs, unroll=4)
        if ci + 2 < _NROWS:
            h[slot] = start_chunk(ci + 2, slot)

    acc = (accs[0] + accs[1]) + (accs[2] + accs[3])
    outv[...] = acc * wv[...]

    @pl.when(c == 0)
    def _():
        pltpu.sync_copy(outv, out0_h.at[s])

    @pl.when(c == 1)
    def _():
        pltpu.sync_copy(outv, out1_h.at[s])


@jax.jit
def _sc_call(advt, orit, idxt, w2d):
    table = pl.pallas_call(
        _table_body,
        out_shape=jax.ShapeDtypeStruct((_B, _K), jnp.float32),
    )(advt, orit)

    mesh = plsc.VectorSubcoreMesh(core_axis_name="c", subcore_axis_name="s")
    f = pl.kernel(
        _worker_body,
        out_type=[jax.ShapeDtypeStruct((_B, _L), jnp.float32),
                  jax.ShapeDtypeStruct((_B, _L), jnp.float32)],
        mesh=mesh,
        compiler_params=pltpu.CompilerParams(needs_layout_passes=False),
        scratch_types=[
            pltpu.VMEM((_K,), jnp.int32),
            pltpu.VMEM((_K,), jnp.int32),
            pltpu.VMEM((_K,), jnp.float32),
            pltpu.VMEM((_L,), jnp.float32),
            pltpu.VMEM((_L,), jnp.float32),
            pltpu.SemaphoreType.DMA,
            pltpu.SemaphoreType.DMA,
            pltpu.SemaphoreType.DMA,
        ],
    )
    p0, p1 = f(table, idxt, w2d)
    return p0 + p1


def kernel(adv_pc, ori_pc, nearest_indices, weights):
    advt = jnp.transpose(adv_pc, (1, 0, 2))
    orit = jnp.transpose(ori_pc, (1, 0, 2))
    idxt = jnp.transpose(nearest_indices.astype(jnp.int32), (0, 2, 1))
    w2d = jnp.broadcast_to(
        (weights.astype(jnp.float32) / _B)[:, None], (_B, _L)
    )
    partials = _sc_call(advt, orit, idxt, w2d)
    return jnp.sum(partials)


# speedup vs baseline: 1.0061x; 1.0061x over previous
import jax
import jax.numpy as jnp
from jax import lax
from jax.experimental import pallas as pl
from jax.experimental.pallas import tpu as pltpu
from jax.experimental.pallas import tpu_sc as plsc

_B = 16
_K = 8192
_KNN = 16
_L = 16
_NROWS = _KNN // 2


def _table_body(advt_ref, orit_ref, out_ref):
    d = advt_ref[...] - orit_ref[...]
    out_ref[...] = jnp.sum(d * d, axis=0)


def _worker_body(tab_h, idx_h, w_h, out_h,
                 idx0, idx1, table, wv, outv, semt, sem0, sem1):
    c = lax.axis_index("c")
    s = lax.axis_index("s")
    b = s
    row = s * 2 + c
    nbase = c * _NROWS

    ht = pltpu.async_copy(tab_h.at[b], table, semt)

    bufs = (idx0, idx1)
    sems = (sem0, sem1)

    def start_chunk(ci, slot):
        return pltpu.async_copy(idx_h.at[b, nbase + ci], bufs[slot], sems[slot])

    h = [start_chunk(0, 0), start_chunk(1, 1)]
    pltpu.sync_copy(w_h.at[b], wv)
    ht.wait()

    zero = jnp.zeros((_L,), jnp.float32)
    accs = (zero, zero, zero, zero)

    def make_gbody(buf):
        def gbody(i, a):
            base = pl.multiple_of(i * 4 * _L, 4 * _L)
            r0 = a[0] + plsc.load_gather(table, [buf[pl.ds(base, _L)]])
            r1 = a[1] + plsc.load_gather(table, [buf[pl.ds(base + _L, _L)]])
            r2 = a[2] + plsc.load_gather(table, [buf[pl.ds(base + 2 * _L, _L)]])
            r3 = a[3] + plsc.load_gather(table, [buf[pl.ds(base + 3 * _L, _L)]])
            return (r0, r1, r2, r3)
        return gbody

    for ci in range(_NROWS):
        slot = ci % 2
        h[slot].wait()
        accs = lax.fori_loop(0, _K // (4 * _L), make_gbody(bufs[slot]),
                             accs, unroll=2)
        if ci + 2 < _NROWS:
            h[slot] = start_chunk(ci + 2, slot)

    acc = (accs[0] + accs[1]) + (accs[2] + accs[3])
    outv[...] = acc * wv[...]
    pltpu.sync_copy(outv, out_h.at[row])


@jax.jit
def _sc_call(advt, orit, idxt, w2d):
    table = pl.pallas_call(
        _table_body,
        out_shape=jax.ShapeDtypeStruct((_B, _K), jnp.float32),
    )(advt, orit)

    mesh = plsc.VectorSubcoreMesh(core_axis_name="c", subcore_axis_name="s")
    f = pl.kernel(
        _worker_body,
        out_type=jax.ShapeDtypeStruct((2 * _B, _L), jnp.float32),
        mesh=mesh,
        compiler_params=pltpu.CompilerParams(needs_layout_passes=False),
        scratch_types=[
            pltpu.VMEM((_K,), jnp.int32),
            pltpu.VMEM((_K,), jnp.int32),
            pltpu.VMEM((_K,), jnp.float32),
            pltpu.VMEM((_L,), jnp.float32),
            pltpu.VMEM((_L,), jnp.float32),
            pltpu.SemaphoreType.DMA,
            pltpu.SemaphoreType.DMA,
            pltpu.SemaphoreType.DMA,
        ],
    )
    return f(table, idxt, w2d)


def kernel(adv_pc, ori_pc, nearest_indices, weights):
    advt = jnp.transpose(adv_pc, (1, 0, 2))
    orit = jnp.transpose(ori_pc, (1, 0, 2))
    idxt = jnp.transpose(nearest_indices.astype(jnp.int32), (0, 2, 1))
    w2d = jnp.broadcast_to(
        (weights.astype(jnp.float32) / _B)[:, None], (_B, _L)
    )
    partials = _sc_call(advt, orit, idxt, w2d)
    return jnp.sum(partials)
